# baseline (speedup 1.0000x reference)
import functools

import jax
import jax.numpy as jnp
from jax import lax
from jax.experimental import pallas as pl
from jax.experimental.pallas import tpu as pltpu
from jax.experimental.pallas import tpu_sc as plsc

N_NODES = 207
NP = 256
IN_F = 32
OUT_F = 8
HID = N_NODES * OUT_F
BATCH = 32
SEQ = 12
M = BATCH * SEQ
E = 1656
EP = 1664
EPS = 1e-5
_ZB = 8192


def _sc_count_body(src_hbm, dst_hbm, out_hbm, src_v, dst_v, idx_v, val_v,
                   zbuf, csh):
    cid = lax.axis_index("c")
    sid = lax.axis_index("s")

    @pl.when(jnp.logical_and(cid == 0, sid == 0))
    def _():
        def _z(i, carry):
            zbuf[pl.ds(i * 16, 16)] = jnp.zeros((16,), jnp.float32)
            return carry
        lax.fori_loop(0, _ZB // 16, _z, 0)

        def _zc(i, carry):
            pltpu.sync_copy(zbuf, csh.at[pl.ds(i * _ZB, _ZB)])
            return carry
        lax.fori_loop(0, (NP * NP) // _ZB, _zc, 0)

        pltpu.sync_copy(src_hbm, src_v)
        pltpu.sync_copy(dst_hbm, dst_v)

        def _ix(i, carry):
            s = src_v[pl.ds(i * 16, 16)]
            d = dst_v[pl.ds(i * 16, 16)]
            lane = lax.iota(jnp.int32, 16) + i * 16
            valid = lane < E
            idx_v[pl.ds(i * 16, 16)] = jnp.where(
                valid, d * NP + s, N_NODES * NP)
            val_v[pl.ds(i * 16, 16)] = jnp.where(valid, 1.0, 0.0)
            return carry
        lax.fori_loop(0, EP // 16, _ix, 0)

        pltpu.sync_copy(val_v, csh.at[idx_v], add=True)

        pltpu.sync_copy(csh, out_hbm)


@functools.cache
def _sc_count_kernel():
    return functools.partial(
        pl.kernel,
        mesh=plsc.VectorSubcoreMesh(core_axis_name="c", subcore_axis_name="s"),
        out_type=jax.ShapeDtypeStruct((NP * NP,), jnp.float32),
        scratch_types=[
            pltpu.VMEM((EP,), jnp.int32),
            pltpu.VMEM((EP,), jnp.int32),
            pltpu.VMEM((EP,), jnp.int32),
            pltpu.VMEM((EP,), jnp.float32),
            pltpu.VMEM((_ZB,), jnp.float32),
            pltpu.VMEM_SHARED((NP * NP,), jnp.float32),
        ],
    )(_sc_count_body)


def _sc_count_matrix(ei):
    src_h = jnp.pad(ei[0], (0, EP - E))
    dst_h = jnp.pad(ei[1], (0, EP - E))
    return _sc_count_kernel()(src_h, dst_h)


def _xw_body(w_sref, x_ref, o_ref):
    for f in range(OUT_F):
        acc = x_ref[0] * w_sref[0, f]
        for i in range(1, IN_F):
            acc = acc + x_ref[i] * w_sref[i, f]
        o_ref[f] = acc


def _xw(x_t3, w_gcn):
    return pl.pallas_call(
        _xw_body,
        out_shape=jax.ShapeDtypeStruct((OUT_F, M, N_NODES), jnp.float32),
        in_specs=[
            pl.BlockSpec(memory_space=pltpu.SMEM),
            pl.BlockSpec(memory_space=pltpu.VMEM),
        ],
        out_specs=pl.BlockSpec(memory_space=pltpu.VMEM),
    )(w_gcn, x_t3)


def _gcn_body(c_ref, xp_ref, bt_ref, o_ref):
    c = c_ref[...]
    deg = jnp.sum(c, axis=1, keepdims=True) + 1.0
    dinv = lax.rsqrt(deg)
    ii = lax.broadcasted_iota(jnp.int32, (NP, NP), 0)
    jj = lax.broadcasted_iota(jnp.int32, (NP, NP), 1)
    mh = (c + jnp.where(ii == jj, 1.0, 0.0)) * dinv
    xs = xp_ref[...] * dinv
    y = jnp.dot(mh, xs, preferred_element_type=jnp.float32)
    o_ref[...] = jnp.maximum(y + bt_ref[...], 0.0)


def _gcn(c, xp, btile):
    return pl.pallas_call(
        _gcn_body,
        out_shape=jax.ShapeDtypeStruct((NP, M * OUT_F), jnp.float32),
    )(c, xp, btile)


def _gi_body(g_ref, w_ref, b_ref, o_ref):
    o_ref[...] = lax.dot_general(
        g_ref[...], w_ref[...], (((1,), (1,)), ((), ())),
        preferred_element_type=jnp.float32) + b_ref[...]


def _gi(gflat, w_ih, b_ih):
    return pl.pallas_call(
        _gi_body,
        out_shape=jax.ShapeDtypeStruct((M, 3 * HID), jnp.float32),
    )(gflat, w_ih, b_ih)


def _gru_body(gi_ref, w_ref, bhh_ref, gam_ref, bet_ref, y_ref, h_ref, wb_ref):
    t = pl.program_id(0)

    @pl.when(t == 0)
    def _():
        wb_ref[...] = w_ref[...].astype(jnp.bfloat16)
        h_ref[...] = jnp.zeros_like(h_ref)

    h = h_ref[...]
    gh = lax.dot_general(
        h.astype(jnp.bfloat16), wb_ref[...], (((1,), (1,)), ((), ())),
        preferred_element_type=jnp.float32) + bhh_ref[...]
    gi = gi_ref[0]
    i_r = gi[:, :HID]
    i_z = gi[:, HID:2 * HID]
    i_n = gi[:, 2 * HID:]
    h_r = gh[:, :HID]
    h_z = gh[:, HID:2 * HID]
    h_n = gh[:, 2 * HID:]
    r = jax.nn.sigmoid(i_r + h_r)
    z = jax.nn.sigmoid(i_z + h_z)
    n = jnp.tanh(i_n + r * h_n)
    hn = (1.0 - z) * n + z * h
    h_ref[...] = hn
    mean = jnp.sum(hn, axis=1, keepdims=True) * (1.0 / HID)
    ex2 = jnp.sum(hn * hn, axis=1, keepdims=True) * (1.0 / HID)
    var = ex2 - mean * mean
    y_ref[:, 0, 0] = ((hn - mean) * lax.rsqrt(var + EPS) * gam_ref[...]
                      + bet_ref[...])


def _gru(gi_t, w_hh, b_hh, gamma, beta):
    return pl.pallas_call(
        _gru_body,
        grid=(SEQ,),
        in_specs=[
            pl.BlockSpec((1, BATCH, 3 * HID), lambda t: (t, 0, 0)),
            pl.BlockSpec((3 * HID, HID), lambda t: (0, 0)),
            pl.BlockSpec((1, 3 * HID), lambda t: (0, 0)),
            pl.BlockSpec((1, HID), lambda t: (0, 0)),
            pl.BlockSpec((1, HID), lambda t: (0, 0)),
        ],
        out_specs=[
            pl.BlockSpec((BATCH, 1, 1, HID), lambda t: (0, t, 0, 0)),
            pl.BlockSpec((BATCH, HID), lambda t: (0, 0)),
        ],
        out_shape=[
            jax.ShapeDtypeStruct((BATCH, SEQ, 1, HID), jnp.float32),
            jax.ShapeDtypeStruct((BATCH, HID), jnp.float32),
        ],
        scratch_shapes=[pltpu.VMEM((3 * HID, HID), jnp.bfloat16)],
    )(gi_t, w_hh, b_hh, gamma, beta)


def kernel(x, edge_index, W_gcn, b_gcn, W_ih, W_hh, b_ih, b_hh, gamma, beta):
    x = x.astype(jnp.float32)
    ei = edge_index.astype(jnp.int32)

    c = _sc_count_matrix(ei).reshape(NP, NP)

    x_t3 = x.reshape(M, N_NODES, IN_F).transpose(2, 0, 1)
    xw8 = _xw(x_t3, W_gcn)
    xp = xw8.transpose(2, 1, 0).reshape(N_NODES, M * OUT_F)
    xp = jnp.pad(xp, ((0, NP - N_NODES), (0, 0)))
    btile = jnp.tile(b_gcn, M)[None, :]

    g = _gcn(c, xp, btile)
    gflat = (g[:N_NODES]
             .reshape(N_NODES, BATCH, SEQ, OUT_F)
             .transpose(2, 1, 0, 3)
             .reshape(M, HID))

    gi = _gi(gflat, W_ih, b_ih[None, :])
    gi_t = gi.reshape(SEQ, BATCH, 3 * HID)

    y, h_t = _gru(gi_t, W_hh, b_hh[None, :], gamma[None, :], beta[None, :])

    x_out = y.reshape(BATCH, SEQ, N_NODES, OUT_F)
    return (x_out, h_t[None, :, :])


# speedup vs baseline: 1.1001x; 1.1001x over previous
import functools

import jax
import jax.numpy as jnp
from jax import lax
from jax.experimental import pallas as pl
from jax.experimental.pallas import tpu as pltpu
from jax.experimental.pallas import tpu_sc as plsc

N_NODES = 207
NP = 256
IN_F = 32
OUT_F = 8
HID = N_NODES * OUT_F
BATCH = 32
SEQ = 12
M = BATCH * SEQ
E = 1656
EP = 1664
EPS = 1e-5
_ZB = 8192


def _sc_count_body(src_hbm, dst_hbm, out_hbm, src_v, dst_v, idx_v, val_v,
                   zbuf, csh):
    cid = lax.axis_index("c")
    sid = lax.axis_index("s")

    @pl.when(jnp.logical_and(cid == 0, sid == 0))
    def _():
        def _z(i, carry):
            zbuf[pl.ds(i * 16, 16)] = jnp.zeros((16,), jnp.float32)
            return carry
        lax.fori_loop(0, _ZB // 16, _z, 0)

        def _zc(i, carry):
            pltpu.sync_copy(zbuf, csh.at[pl.ds(i * _ZB, _ZB)])
            return carry
        lax.fori_loop(0, (NP * NP) // _ZB, _zc, 0)

        pltpu.sync_copy(src_hbm, src_v)
        pltpu.sync_copy(dst_hbm, dst_v)

        def _ix(i, carry):
            s = src_v[pl.ds(i * 16, 16)]
            d = dst_v[pl.ds(i * 16, 16)]
            lane = lax.iota(jnp.int32, 16) + i * 16
            valid = lane < E
            idx_v[pl.ds(i * 16, 16)] = jnp.where(
                valid, d * NP + s, N_NODES * NP)
            val_v[pl.ds(i * 16, 16)] = jnp.where(valid, 1.0, 0.0)
            return carry
        lax.fori_loop(0, EP // 16, _ix, 0)

        pltpu.sync_copy(val_v, csh.at[idx_v], add=True)

        pltpu.sync_copy(csh, out_hbm)


@functools.cache
def _sc_count_kernel():
    return functools.partial(
        pl.kernel,
        mesh=plsc.VectorSubcoreMesh(core_axis_name="c", subcore_axis_name="s"),
        out_type=jax.ShapeDtypeStruct((NP * NP,), jnp.float32),
        scratch_types=[
            pltpu.VMEM((EP,), jnp.int32),
            pltpu.VMEM((EP,), jnp.int32),
            pltpu.VMEM((EP,), jnp.int32),
            pltpu.VMEM((EP,), jnp.float32),
            pltpu.VMEM((_ZB,), jnp.float32),
            pltpu.VMEM_SHARED((NP * NP,), jnp.float32),
        ],
    )(_sc_count_body)


def _sc_count_matrix(ei):
    src_h = jnp.pad(ei[0], (0, EP - E))
    dst_h = jnp.pad(ei[1], (0, EP - E))
    return _sc_count_kernel()(src_h, dst_h)


def _xw_body(w_sref, x_ref, o_ref):
    for f in range(OUT_F):
        acc = x_ref[0] * w_sref[0, f]
        for i in range(1, IN_F):
            acc = acc + x_ref[i] * w_sref[i, f]
        o_ref[f] = acc.astype(jnp.bfloat16)


def _xw(x_t3, w_gcn):
    return pl.pallas_call(
        _xw_body,
        out_shape=jax.ShapeDtypeStruct((OUT_F, M, N_NODES), jnp.bfloat16),
        in_specs=[
            pl.BlockSpec(memory_space=pltpu.SMEM),
            pl.BlockSpec(memory_space=pltpu.VMEM),
        ],
        out_specs=pl.BlockSpec(memory_space=pltpu.VMEM),
    )(w_gcn, x_t3)


def _gcn_body(c_ref, xp_ref, bt_ref, o_ref):
    c = c_ref[...]
    deg = jnp.sum(c, axis=1, keepdims=True) + 1.0
    dinv = lax.rsqrt(deg)
    ii = lax.broadcasted_iota(jnp.int32, (NP, NP), 0)
    jj = lax.broadcasted_iota(jnp.int32, (NP, NP), 1)
    mh = (c + jnp.where(ii == jj, 1.0, 0.0)) * dinv
    xs = xp_ref[...].astype(jnp.float32) * dinv
    y = jnp.dot(mh, xs, preferred_element_type=jnp.float32)
    o_ref[...] = jnp.maximum(y + bt_ref[...], 0.0).astype(jnp.bfloat16)


def _gcn(c, xp, btile):
    return pl.pallas_call(
        _gcn_body,
        out_shape=jax.ShapeDtypeStruct((NP, M * OUT_F), jnp.bfloat16),
    )(c, xp, btile)


def _gi_body(g_ref, w_ref, b_ref, o_ref):
    o_ref[0] = (lax.dot_general(
        g_ref[...], w_ref[0].astype(jnp.bfloat16), (((1,), (1,)), ((), ())),
        preferred_element_type=jnp.float32) + b_ref[0]).astype(jnp.bfloat16)


def _gi(gflat, w3, b3):
    return pl.pallas_call(
        _gi_body,
        grid=(3,),
        in_specs=[
            pl.BlockSpec((M, HID), lambda j: (0, 0)),
            pl.BlockSpec((1, HID, HID), lambda j: (j, 0, 0)),
            pl.BlockSpec((1, 1, HID), lambda j: (j, 0, 0)),
        ],
        out_specs=pl.BlockSpec((1, M, HID), lambda j: (j, 0, 0)),
        out_shape=jax.ShapeDtypeStruct((3, M, HID), jnp.bfloat16),
    )(gflat, w3, b3)


def _gru_body(gi_ref, w_ref, bhh_ref, gam_ref, bet_ref, y_ref, h_ref, wb_ref):
    t = pl.program_id(0)

    @pl.when(t == 0)
    def _():
        wb_ref[...] = w_ref[...].astype(jnp.bfloat16)
        h_ref[...] = jnp.zeros_like(h_ref)

    h = h_ref[...]
    hb = h.astype(jnp.bfloat16)
    dn = (((1,), (1,)), ((), ()))
    f32 = jnp.float32
    gh_r = lax.dot_general(hb, wb_ref[0], dn,
                           preferred_element_type=f32) + bhh_ref[0]
    gh_z = lax.dot_general(hb, wb_ref[1], dn,
                           preferred_element_type=f32) + bhh_ref[1]
    gh_n = lax.dot_general(hb, wb_ref[2], dn,
                           preferred_element_type=f32) + bhh_ref[2]
    r = jax.nn.sigmoid(gi_ref[0, 0].astype(f32) + gh_r)
    z = jax.nn.sigmoid(gi_ref[1, 0].astype(f32) + gh_z)
    n = jnp.tanh(gi_ref[2, 0].astype(f32) + r * gh_n)
    hn = (1.0 - z) * n + z * h
    h_ref[...] = hn
    mean = jnp.sum(hn, axis=1, keepdims=True) * (1.0 / HID)
    ex2 = jnp.sum(hn * hn, axis=1, keepdims=True) * (1.0 / HID)
    var = ex2 - mean * mean
    y_ref[:, 0, 0] = ((hn - mean) * lax.rsqrt(var + EPS) * gam_ref[...]
                      + bet_ref[...])


def _gru(gi4, w3, bhh3, gamma, beta):
    return pl.pallas_call(
        _gru_body,
        grid=(SEQ,),
        in_specs=[
            pl.BlockSpec((3, 1, BATCH, HID), lambda t: (0, t, 0, 0)),
            pl.BlockSpec((3, HID, HID), lambda t: (0, 0, 0)),
            pl.BlockSpec((3, 1, HID), lambda t: (0, 0, 0)),
            pl.BlockSpec((1, HID), lambda t: (0, 0)),
            pl.BlockSpec((1, HID), lambda t: (0, 0)),
        ],
        out_specs=[
            pl.BlockSpec((BATCH, 1, 1, HID), lambda t: (0, t, 0, 0)),
            pl.BlockSpec((BATCH, HID), lambda t: (0, 0)),
        ],
        out_shape=[
            jax.ShapeDtypeStruct((BATCH, SEQ, 1, HID), jnp.float32),
            jax.ShapeDtypeStruct((BATCH, HID), jnp.float32),
        ],
        scratch_shapes=[pltpu.VMEM((3, HID, HID), jnp.bfloat16)],
    )(gi4, w3, bhh3, gamma, beta)


def kernel(x, edge_index, W_gcn, b_gcn, W_ih, W_hh, b_ih, b_hh, gamma, beta):
    x = x.astype(jnp.float32)
    ei = edge_index.astype(jnp.int32)

    c = _sc_count_matrix(ei).reshape(NP, NP)

    x_t3 = x.reshape(M, N_NODES, IN_F).transpose(2, 0, 1)
    xw8 = _xw(x_t3, W_gcn)
    xp = xw8.transpose(2, 1, 0).reshape(N_NODES, M * OUT_F)
    xp = jnp.pad(xp, ((0, NP - N_NODES), (0, 0)))
    btile = jnp.tile(b_gcn, M)[None, :]

    g = _gcn(c, xp, btile)
    gflat = (g[:N_NODES]
             .reshape(N_NODES, BATCH, SEQ, OUT_F)
             .transpose(2, 1, 0, 3)
             .reshape(M, HID))

    gi3 = _gi(gflat, W_ih.reshape(3, HID, HID),
              b_ih.reshape(3, 1, HID))
    gi4 = gi3.reshape(3, SEQ, BATCH, HID)

    y, h_t = _gru(gi4, W_hh.reshape(3, HID, HID),
                  b_hh.reshape(3, 1, HID), gamma[None, :], beta[None, :])

    x_out = y.reshape(BATCH, SEQ, N_NODES, OUT_F)
    return (x_out, h_t[None, :, :])
